# baseline (speedup 1.0000x reference)
import functools

import jax
import jax.numpy as jnp
from jax import lax
from jax.experimental import pallas as pl
from jax.experimental.pallas import tpu as pltpu
from jax.experimental.pallas import tpu_sc as plsc

_ROWS = 16384
_COLS = 256
_NOUT = 8
_W = 32
_NC = 2
_NS = 16
_NW = _NC * _NS
_RPW = _ROWS // _NW
_RC = 128


def _sc_split_body(x_hbm, *rest):
    outs = rest[:_NOUT]
    sem = rest[_NOUT]
    wid = lax.axis_index("s") * _NC + lax.axis_index("c")
    base = wid * _RPW
    cps = []
    for i in range(_NOUT):
        cp = pltpu.make_async_copy(
            x_hbm.at[pl.ds(base, _RPW), pl.ds(i * _W, _W)],
            outs[i].at[pl.ds(base, _RPW)], sem)
        cp.start()
        cps.append(cp)
    for cp in cps:
        cp.wait()


@jax.jit
def kernel(x):
    mesh = plsc.VectorSubcoreMesh(core_axis_name="c", subcore_axis_name="s")
    out_type = tuple(
        jax.ShapeDtypeStruct((_ROWS, _W), jnp.float32) for _ in range(_NOUT))
    scratch = [
        pltpu.SemaphoreType.DMA,
    ]
    f = pl.kernel(
        _sc_split_body,
        out_type=out_type,
        mesh=mesh,
        scratch_types=scratch,
        compiler_params=pltpu.CompilerParams(use_tc_tiling_on_sc=False),
    )
    return f(x)


# speedup vs baseline: 4.6368x; 4.6368x over previous
import functools

import jax
import jax.numpy as jnp
from jax import lax
from jax.experimental import pallas as pl
from jax.experimental.pallas import tpu as pltpu
from jax.experimental.pallas import tpu_sc as plsc

_ROWS = 16384
_COLS = 256
_NOUT = 8
_W = 32
_NC = 2
_NS = 16
_NW = _NC * _NS
_RPW = _ROWS // _NW
_RC = 128


_RH = 256
_NB = 8
_NTASK = (_RPW // _RH) * _NOUT


def _task(base, t):
    i, h = t % _NOUT, t // _NOUT
    return i, base + h * _RH


def _sc_split_body(x_hbm, *rest):
    outs = rest[:_NOUT]
    bufs = rest[_NOUT:_NOUT + _NB]
    isems = rest[_NOUT + _NB:_NOUT + 2 * _NB]
    osems = rest[_NOUT + 2 * _NB:]
    wid = lax.axis_index("s") * _NC + lax.axis_index("c")
    base = wid * _RPW

    def start_in(t, b):
        i, rb = _task(base, t)
        pltpu.make_async_copy(
            x_hbm.at[pl.ds(rb, _RH), pl.ds(i * _W, _W)], bufs[b],
            isems[b]).start()

    def wait_in(t, b):
        i, rb = _task(base, t)
        pltpu.make_async_copy(
            x_hbm.at[pl.ds(rb, _RH), pl.ds(i * _W, _W)], bufs[b],
            isems[b]).wait()

    def make_out(t, b):
        i, rb = _task(base, t)
        return pltpu.make_async_copy(
            bufs[b], outs[i].at[pl.ds(rb, _RH)], osems[b])

    for t in range(_NB):
        start_in(t, t)
    for t in range(_NTASK):
        b = t % _NB
        wait_in(t, b)
        make_out(t, b).start()
        if t + _NB < _NTASK:
            make_out(t, b).wait()
            start_in(t + _NB, b)
    for t in range(_NTASK - _NB, _NTASK):
        make_out(t, t % _NB).wait()


@jax.jit
def kernel(x):
    mesh = plsc.VectorSubcoreMesh(core_axis_name="c", subcore_axis_name="s")
    out_type = tuple(
        jax.ShapeDtypeStruct((_ROWS, _W), jnp.float32) for _ in range(_NOUT))
    scratch = (
        [pltpu.VMEM((_RH, _W), jnp.float32) for _ in range(_NB)]
        + [pltpu.SemaphoreType.DMA for _ in range(2 * _NB)])
    f = pl.kernel(
        _sc_split_body,
        out_type=out_type,
        mesh=mesh,
        scratch_types=scratch,
        compiler_params=pltpu.CompilerParams(use_tc_tiling_on_sc=False),
    )
    return f(x)


# speedup vs baseline: 4.9443x; 1.0663x over previous
import functools

import jax
import jax.numpy as jnp
from jax import lax
from jax.experimental import pallas as pl
from jax.experimental.pallas import tpu as pltpu
from jax.experimental.pallas import tpu_sc as plsc

_ROWS = 16384
_COLS = 256
_NOUT = 8
_W = 32
_L = 16
_NC = 2
_NS = 16
_NW = _NC * _NS
_RPW = _ROWS // _NW
_RC = 64
_NCH = _RPW // _RC


def _sc_split_body(x_hbm, *rest):
    outs = rest[:_NOUT]
    inbufs = rest[_NOUT:_NOUT + 2]
    obufs = rest[_NOUT + 2:_NOUT + 2 + _NOUT]
    isems = rest[_NOUT + 2 + _NOUT:_NOUT + 2 + _NOUT + 2]
    osem = rest[-1]
    wid = lax.axis_index("s") * _NC + lax.axis_index("c")
    base = wid * _RPW

    def in_cp(h, b):
        return pltpu.make_async_copy(
            x_hbm.at[pl.ds(base + h * _RC, _RC)], inbufs[b], isems[b])

    in_cp(0, 0).start()
    for h in range(_NCH):
        b = h % 2
        if h + 1 < _NCH:
            in_cp(h + 1, 1 - b).start()
        in_cp(h, b).wait()
        if h > 0:
            for i in range(_NOUT):
                pltpu.make_async_copy(
                    obufs[i], outs[i].at[pl.ds(base + (h - 1) * _RC, _RC)],
                    osem).wait()
        ib = inbufs[b]

        def split_row(r, _):
            for i in range(_NOUT):
                for c in range(_W // _L):
                    obufs[i][r, pl.ds(c * _L, _L)] = (
                        ib[r, pl.ds(i * _W + c * _L, _L)])
            return _

        lax.fori_loop(0, _RC, split_row, 0, unroll=4)
        for i in range(_NOUT):
            pltpu.make_async_copy(
                obufs[i], outs[i].at[pl.ds(base + h * _RC, _RC)], osem).start()
    for i in range(_NOUT):
        pltpu.make_async_copy(
            obufs[i], outs[i].at[pl.ds(base + (_NCH - 1) * _RC, _RC)],
            osem).wait()


@jax.jit
def kernel(x):
    mesh = plsc.VectorSubcoreMesh(core_axis_name="c", subcore_axis_name="s")
    out_type = tuple(
        jax.ShapeDtypeStruct((_ROWS, _W), jnp.float32) for _ in range(_NOUT))
    scratch = (
        [pltpu.VMEM((_RC, _COLS), jnp.float32) for _ in range(2)]
        + [pltpu.VMEM((_RC, _W), jnp.float32) for _ in range(_NOUT)]
        + [pltpu.SemaphoreType.DMA for _ in range(3)])
    f = pl.kernel(
        _sc_split_body,
        out_type=out_type,
        mesh=mesh,
        scratch_types=scratch,
    )
    return f(x)


# speedup vs baseline: 12.8046x; 2.5898x over previous
import functools

import jax
import jax.numpy as jnp
from jax import lax
from jax.experimental import pallas as pl
from jax.experimental.pallas import tpu as pltpu
from jax.experimental.pallas import tpu_sc as plsc

_ROWS = 16384
_NOUT = 8
_W = 32
_NC = 2
_NS = 16
_NW = _NC * _NS
_CC = _ROWS // _NW


def _sc_copy_body(xt_hbm, *rest):
    outs = rest[:_NOUT]
    bufs = rest[_NOUT:_NOUT + 2]
    isems = rest[_NOUT + 2:_NOUT + 4]
    osems = rest[_NOUT + 4:]
    wid = lax.axis_index("s") * _NC + lax.axis_index("c")
    c0 = wid * _CC

    def in_cp(i, b):
        return pltpu.make_async_copy(
            xt_hbm.at[pl.ds(i * _W, _W), pl.ds(c0, _CC)], bufs[b], isems[b])

    def out_cp(i, b):
        return pltpu.make_async_copy(
            bufs[b], outs[i].at[:, pl.ds(c0, _CC)], osems[b])

    in_cp(0, 0).start()
    for i in range(_NOUT):
        b = i % 2
        if i + 1 < _NOUT:
            if i >= 1:
                out_cp(i - 1, 1 - b).wait()
            in_cp(i + 1, 1 - b).start()
        in_cp(i, b).wait()
        out_cp(i, b).start()
    out_cp(_NOUT - 2, (_NOUT - 2) % 2).wait()
    out_cp(_NOUT - 1, (_NOUT - 1) % 2).wait()


def _sc_split_t(xt):
    mesh = plsc.VectorSubcoreMesh(core_axis_name="c", subcore_axis_name="s")
    out_type = tuple(
        jax.ShapeDtypeStruct((_W, _ROWS), jnp.float32) for _ in range(_NOUT))
    scratch = (
        [pltpu.VMEM((_W, _CC), jnp.float32) for _ in range(2)]
        + [pltpu.SemaphoreType.DMA for _ in range(4)])
    return pl.kernel(
        _sc_copy_body,
        out_type=out_type,
        mesh=mesh,
        scratch_types=scratch,
    )(xt)


@jax.jit
def kernel(x):
    yts = _sc_split_t(x.T)
    return tuple(yt.T for yt in yts)
